# baseline (speedup 1.0000x reference)
import jax
import jax.numpy as jnp
from jax.experimental import pallas as pl
from jax.experimental.pallas import tpu as pltpu

_CHUNKS = (128, 512, 1408)


def _cluster_kernel(data_hbm, cent_ref, out_ref, dbuf, buf, in_sems, out_sems):
    base = 0
    in_copies = []
    for i, bn in enumerate(_CHUNKS):
        cp = pltpu.make_async_copy(
            data_hbm.at[pl.ds(base, bn), :], dbuf.at[pl.ds(base, bn), :],
            in_sems.at[i])
        cp.start()
        in_copies.append(cp)
        base += bn

    cent = cent_ref[:, :]
    k = cent.shape[0]
    ccp1 = jnp.sum(cent * cent, axis=1) + 1.0
    ca = jnp.concatenate(
        [cent * -2.0, ccp1[:, None], jnp.ones((k, 1), jnp.float32)], axis=1)

    base = 0
    out_copies = []
    for i, bn in enumerate(_CHUNKS):
        in_copies[i].wait()
        data = dbuf[pl.ds(base, bn), :]
        xx = jnp.sum(data * data, axis=1)
        xa = jnp.concatenate(
            [data, jnp.ones((bn, 1), jnp.float32), xx[:, None]], axis=1)
        u = jax.lax.dot_general(
            ca, xa, (((1,), (1,)), ((), ())),
            preferred_element_type=jnp.float32)
        t = 1.0 / (u * u)
        s = jnp.sum(t, axis=0)
        buf[:, pl.ds(base, bn)] = t * (1.0 / s)[None, :]
        cp = pltpu.make_async_copy(
            buf.at[:, pl.ds(base, bn)], out_ref.at[:, pl.ds(base, bn)],
            out_sems.at[i])
        cp.start()
        out_copies.append(cp)
        base += bn
    for cp in out_copies:
        cp.wait()


def kernel(data, centroids):
    n, d = data.shape
    k, _ = centroids.shape
    nc = len(_CHUNKS)
    return pl.pallas_call(
        _cluster_kernel,
        in_specs=[
            pl.BlockSpec(memory_space=pltpu.MemorySpace.HBM),
            pl.BlockSpec(memory_space=pltpu.MemorySpace.VMEM),
        ],
        out_specs=pl.BlockSpec(memory_space=pltpu.MemorySpace.HBM),
        out_shape=jax.ShapeDtypeStruct((k, n), jnp.float32),
        scratch_shapes=[
            pltpu.VMEM((n, d), jnp.float32),
            pltpu.VMEM((k, n), jnp.float32),
            pltpu.SemaphoreType.DMA((nc,)),
            pltpu.SemaphoreType.DMA((nc,)),
        ],
    )(data, centroids)


# speedup vs baseline: 1.1402x; 1.1402x over previous
import jax
import jax.numpy as jnp
from jax.experimental import pallas as pl

_BN = 1024


def _cluster_kernel(data_ref, cent_ref, out_ref):
    data = data_ref[:, :]
    cent = cent_ref[:, :]
    xx = jnp.sum(data * data, axis=1)
    ccp1 = jnp.sum(cent * cent, axis=1) + 1.0
    bn = data.shape[0]
    k = cent.shape[0]
    ca = jnp.concatenate(
        [cent * -2.0, ccp1[:, None], jnp.ones((k, 1), jnp.float32)], axis=1)
    xa = jnp.concatenate(
        [data, jnp.ones((bn, 1), jnp.float32), xx[:, None]], axis=1)
    u = jax.lax.dot_general(
        ca, xa, (((1,), (1,)), ((), ())),
        preferred_element_type=jnp.float32)
    t = 1.0 / (u * u)
    s = jnp.sum(t, axis=0)
    out_ref[:, :] = t * (1.0 / s)[None, :]


def kernel(data, centroids):
    n, d = data.shape
    k, _ = centroids.shape
    return pl.pallas_call(
        _cluster_kernel,
        grid=(n // _BN,),
        in_specs=[
            pl.BlockSpec((_BN, d), lambda i: (i, 0)),
            pl.BlockSpec((k, d), lambda i: (0, 0)),
        ],
        out_specs=pl.BlockSpec((k, _BN), lambda i: (0, i)),
        out_shape=jax.ShapeDtypeStruct((k, n), jnp.float32),
    )(data, centroids)


# speedup vs baseline: 1.1403x; 1.0001x over previous
import jax
import jax.numpy as jnp
from jax.experimental import pallas as pl
from jax.experimental.pallas import tpu as pltpu

_BN = 1024


def _cluster_kernel(data_ref, cent_ref, out_ref):
    data = data_ref[:, :]
    cent = cent_ref[:, :]
    xx = jnp.sum(data * data, axis=1)
    ccp1 = jnp.sum(cent * cent, axis=1) + 1.0
    bn = data.shape[0]
    k = cent.shape[0]
    ca = jnp.concatenate(
        [cent * -2.0, ccp1[:, None], jnp.ones((k, 1), jnp.float32)], axis=1)
    xa = jnp.concatenate(
        [data, jnp.ones((bn, 1), jnp.float32), xx[:, None]], axis=1)
    u = jax.lax.dot_general(
        ca, xa, (((1,), (1,)), ((), ())),
        preferred_element_type=jnp.float32)
    t = 1.0 / (u * u)
    s = jnp.sum(t, axis=0)
    out_ref[:, :] = t * (1.0 / s)[None, :]


def kernel(data, centroids):
    n, d = data.shape
    k, _ = centroids.shape
    return pl.pallas_call(
        _cluster_kernel,
        grid=(n // _BN,),
        in_specs=[
            pl.BlockSpec((_BN, d), lambda i: (i, 0)),
            pl.BlockSpec((k, d), lambda i: (0, 0)),
        ],
        out_specs=pl.BlockSpec((k, _BN), lambda i: (0, i)),
        out_shape=jax.ShapeDtypeStruct((k, n), jnp.float32),
        compiler_params=pltpu.CompilerParams(
            dimension_semantics=("parallel",)),
    )(data, centroids)
